# baseline (speedup 1.0000x reference)
import functools

import jax
import jax.numpy as jnp
from jax.experimental import pallas as pl
from jax.experimental.pallas import tpu as pltpu

B = 1024
V = 100000
W = 128
NBUF = 8


def _body(lbl_ref, probs_ref, out_ref, ring, acc, sems):
    def issue(k, slot):
        lbl = lbl_ref[k]
        cb = (lbl // W) * W
        return pltpu.make_async_copy(
            probs_ref.at[pl.ds(k, 1), pl.ds(cb, W)],
            ring.at[pl.ds(slot, 1), :],
            sems.at[slot],
        )

    acc[...] = jnp.zeros((1, W), jnp.float32)
    lane = jax.lax.broadcasted_iota(jnp.int32, (1, W), 1)

    for k in range(NBUF):
        issue(k, k).start()

    def step(k, _):
        slot = jax.lax.rem(k, NBUF)
        issue(k, slot).wait()
        col = lbl_ref[k] % W
        row = ring[pl.ds(slot, 1), :]
        acc[...] += jnp.where(lane == col, row, 0.0)

        @pl.when(k + NBUF < B)
        def _next():
            issue(k + NBUF, slot).start()

        return 0

    jax.lax.fori_loop(0, B, step, 0)
    out_ref[0, 0] = -jnp.sum(acc[...]) / B


@jax.jit
def _tc_loss(probs, labels):
    out = pl.pallas_call(
        _body,
        grid=(1,),
        in_specs=[
            pl.BlockSpec(memory_space=pltpu.SMEM),
            pl.BlockSpec(memory_space=pl.ANY),
        ],
        out_specs=pl.BlockSpec(memory_space=pltpu.SMEM),
        out_shape=jax.ShapeDtypeStruct((1, 1), jnp.float32),
        scratch_shapes=[
            pltpu.VMEM((NBUF, W), jnp.float32),
            pltpu.VMEM((1, W), jnp.float32),
            pltpu.SemaphoreType.DMA((NBUF,)),
        ],
    )(labels, probs)
    return out[0, 0]


def kernel(probs, true_label):
    return _tc_loss(probs, true_label.astype(jnp.int32))


# speedup vs baseline: 1.1022x; 1.1022x over previous
import functools

import jax
import jax.numpy as jnp
from jax.experimental import pallas as pl
from jax.experimental.pallas import tpu as pltpu

B = 1024
V = 100000
W = 128
R = 16
STEPS = B // R


def _body(lbl_ref, *refs):
    i = pl.program_id(0)
    row_refs = refs[:R]
    out_ref = refs[R]
    acc = refs[R + 1]

    @pl.when(i == 0)
    def _init():
        acc[...] = jnp.zeros((8, W), jnp.float32)

    lane = jax.lax.broadcasted_iota(jnp.int32, (8, W), 1)
    sub = jax.lax.broadcasted_iota(jnp.int32, (8, W), 0)
    a = acc[...]
    for r in range(R):
        col = lbl_ref[i * R + r] % W
        a = a + jnp.where((sub == r % 8) & (lane == col),
                          row_refs[r][...], 0.0)
    acc[...] = a

    @pl.when(i == STEPS - 1)
    def _fin():
        out_ref[0, 0] = -jnp.sum(acc[...]) / B


def _mk_spec(r):
    def index_map(i, lbl_ref):
        return (2 * i + r // 8, lbl_ref[i * R + r] // W)
    return pl.BlockSpec((8, W), index_map)


@jax.jit
def _tc_loss(probs, labels):
    out = pl.pallas_call(
        _body,
        grid_spec=pltpu.PrefetchScalarGridSpec(
            num_scalar_prefetch=1,
            grid=(STEPS,),
            in_specs=[_mk_spec(r) for r in range(R)],
            out_specs=pl.BlockSpec(memory_space=pltpu.SMEM),
            scratch_shapes=[pltpu.VMEM((8, W), jnp.float32)],
        ),
        out_shape=jax.ShapeDtypeStruct((1, 1), jnp.float32),
    )(labels, *([probs] * R))
    return out[0, 0]


def kernel(probs, true_label):
    return _tc_loss(probs, true_label.astype(jnp.int32))


# speedup vs baseline: 1.1602x; 1.0526x over previous
import functools

import jax
import jax.numpy as jnp
from jax import lax
from jax.experimental import pallas as pl
from jax.experimental.pallas import tpu as pltpu
from jax.experimental.pallas import tpu_sc as plsc

B = 1024
V = 100000
NC = 2
NS = 16
L = 16
NW = NC * NS
PER_W = B // NW
CH = 128


def _body(probs_hbm, lbl_hbm, out_hbm, lbl_v, lbl_s, idx_v, val_v, flat_v,
          pick_v, part_v, red_v, slots_v, shared_slots, shared_red, sem):
    c = lax.axis_index("c")
    s = lax.axis_index("s")
    wid = s * NC + c
    base = wid * PER_W

    pltpu.sync_copy(lbl_hbm.at[pl.ds(base, PER_W)], lbl_v)
    for j in range(PER_W // L):
        lbls = lbl_v[pl.ds(j * L, L)]
        for t in range(L):
            lbl_s[j * L + t] = lbls[t]

    copies = []
    for k in range(PER_W):
        coff = (lbl_s[k] // CH) * CH
        copies.append(
            pltpu.async_copy(
                probs_hbm.at[pl.ds(base + (k // 8) * 8, 8),
                             pl.ds(coff, CH)],
                val_v.at[pl.ds(k * 8, 8), :], sem))
    for cp in copies:
        cp.wait()

    fbase = s * PER_W * CH
    for k in range(PER_W):
        pltpu.sync_copy(val_v.at[k * 8 + k % 8, :],
                        flat_v.at[pl.ds(fbase + k * CH, CH)])
    lane = lax.iota(jnp.int32, L)
    for j in range(PER_W // L):
        lbls = lbl_v[pl.ds(j * L, L)]
        idx_v[pl.ds(j * L, L)] = fbase + (j * L + lane) * CH + lbls % CH
    pltpu.async_copy(flat_v.at[idx_v], pick_v, sem).wait()

    acc = pick_v[pl.ds(0, L)]
    for j in range(1, PER_W // L):
        acc = acc + pick_v[pl.ds(j * L, L)]
    part_v[...] = acc * (-1.0 / B)

    pltpu.sync_copy(part_v, shared_slots.at[pl.ds(s * L, L)])
    plsc.subcore_barrier()

    @pl.when(s == 0)
    def _finish():
        pltpu.sync_copy(shared_slots, slots_v)
        acc = slots_v[pl.ds(0, L)]
        for r in range(1, NS):
            acc = acc + slots_v[pl.ds(r * L, L)]
        red_v[...] = jnp.zeros((L,), jnp.float32)
        pltpu.sync_copy(red_v, shared_red)
        red_v[...] = acc
        pltpu.sync_copy(red_v, shared_red.at[jnp.zeros((L,), jnp.int32)],
                        add=True)
        pltpu.sync_copy(shared_red, red_v)
        pltpu.sync_copy(red_v, out_hbm.at[c])


@jax.jit
def _sc_loss(probs, labels):
    out = pl.kernel(
        _body,
        out_type=jax.ShapeDtypeStruct((NC, L), jnp.float32),
        mesh=plsc.VectorSubcoreMesh(core_axis_name="c", subcore_axis_name="s"),
        compiler_params=pltpu.CompilerParams(use_tc_tiling_on_sc=True),
        scratch_types=[
            pltpu.VMEM((PER_W,), jnp.int32),
            pltpu.SMEM((PER_W,), jnp.int32),
            pltpu.VMEM((PER_W,), jnp.int32),
            pltpu.VMEM((PER_W * 8, CH), jnp.float32),
            pltpu.VMEM_SHARED((NS * PER_W * CH,), jnp.float32),
            pltpu.VMEM((PER_W,), jnp.float32),
            pltpu.VMEM((L,), jnp.float32),
            pltpu.VMEM((L,), jnp.float32),
            pltpu.VMEM((NS * L,), jnp.float32),
            pltpu.VMEM_SHARED((NS * L,), jnp.float32),
            pltpu.VMEM_SHARED((L,), jnp.float32),
            pltpu.SemaphoreType.DMA,
        ],
    )(probs, labels)
    return out[0, 0] + out[1, 0]


def kernel(probs, true_label):
    return _sc_loss(probs, true_label.astype(jnp.int32))
